# baseline (speedup 1.0000x reference)
import functools

import jax
import jax.numpy as jnp
from jax import lax
from jax.experimental import pallas as pl
from jax.experimental.pallas import tpu as pltpu
from jax.experimental.pallas import tpu_sc as plsc

N = 10000
E = 320000
D_IN = 128
D_H = 64

NC = 2
NS = 16
NW = NC * NS

NPAD = 10240
ECH = 128
CPT = 80
EPAD = ECH * CPT * NW
C0 = 80
C1 = 2 * CPT - C0
CMX = max(C0, C1)
DEGW = 16
RPT = NPAD // NS
W_SC = D_H // 2

_mesh = plsc.VectorSubcoreMesh(core_axis_name="c", subcore_axis_name="s")
_sc_params = pltpu.CompilerParams(use_tc_tiling_on_sc=False, needs_layout_passes=False)


def _edge_body(h, comb2d, out, acc_sh, h_sh, combbuf, srcbuf, dstbuf, rows, zbuf, sem):
    c = lax.axis_index("c")
    s = lax.axis_index("s")
    zero16 = jnp.zeros((16,), jnp.float32)

    def fill_z(i, _):
        zbuf[i, pl.ds(0, 16)] = zero16
        zbuf[i, pl.ds(16, 16)] = zero16
        return 0

    lax.fori_loop(0, RPT // 2, fill_z, 0)
    pltpu.sync_copy(zbuf, acc_sh.at[pl.ds(s * RPT, RPT // 2)])
    pltpu.sync_copy(zbuf, acc_sh.at[pl.ds(s * RPT + RPT // 2, RPT // 2)])
    for half in range(2):
        off = s * RPT + half * (RPT // 2)
        pltpu.sync_copy(h.at[pl.ds(off, RPT // 2)], zbuf)
        pltpu.sync_copy(zbuf, h_sh.at[pl.ds(off, RPT // 2)])

    cpt = jnp.where(c == 0, C0, C1)

    @pl.when(c == 0)
    def _load_idx0():
        pltpu.sync_copy(comb2d.at[pl.ds(s * C0, C0)], combbuf.at[pl.ds(0, C0)])

    @pl.when(c == 1)
    def _load_idx1():
        pltpu.sync_copy(comb2d.at[pl.ds(NS * C0 + s * C1, C1)], combbuf.at[pl.ds(0, C1)])

    def extract(i, _):
        for k in range(8):
            v = plsc.bitcast(combbuf[i, pl.ds(16 * k, 16)], jnp.int32)
            srcbuf[i, pl.ds(16 * k, 16)] = lax.bitwise_and(v, 0xFFFF)
            dstbuf[i, pl.ds(16 * k, 16)] = lax.shift_right_logical(v, 16)
        return 0

    lax.fori_loop(0, cpt, extract, 0)
    plsc.subcore_barrier()

    pltpu.async_copy(h_sh.at[srcbuf.at[0]], rows.at[0], sem.at[0])

    def chunk(j, _):
        p = lax.rem(j, 2)

        @pl.when(j + 1 < cpt)
        def _start_next():
            pltpu.async_copy(h_sh.at[srcbuf.at[j + 1]], rows.at[1 - p], sem.at[1 - p])

        pltpu.make_async_copy(h_sh.at[srcbuf.at[j]], rows.at[p], sem.at[p]).wait()
        pltpu.sync_copy(rows.at[p], acc_sh.at[dstbuf.at[j]], add=True)
        return 0

    lax.fori_loop(0, cpt, chunk, 0)
    plsc.subcore_barrier()

    pltpu.sync_copy(acc_sh.at[pl.ds(s * RPT, RPT)], out.at[c, pl.ds(s * RPT, RPT)])


_edge_kernel = functools.partial(
    pl.kernel,
    out_type=jax.ShapeDtypeStruct((NC, NPAD, W_SC), jnp.float32),
    mesh=_mesh,
    scratch_types=[
        pltpu.VMEM_SHARED((NPAD, W_SC), jnp.float32),
        pltpu.VMEM_SHARED((NPAD, W_SC), jnp.float32),
        pltpu.VMEM((CMX, ECH), jnp.float32),
        pltpu.VMEM((CMX, ECH), jnp.int32),
        pltpu.VMEM((CMX, ECH), jnp.int32),
        pltpu.VMEM((2, ECH, W_SC), jnp.float32),
        pltpu.VMEM((RPT // 2, W_SC), jnp.float32),
        pltpu.SemaphoreType.DMA((2,)),
    ],
    compiler_params=_sc_params,
)(_edge_body)


def _dis_body(p0_ref, p1_ref, o_ref):
    deg = p0_ref[...] + p1_ref[...] + 1.0
    o_ref[...] = lax.rsqrt(deg)


def _mm0_body(x_ref, w_ref, d_ref, o_ref):
    h = jnp.dot(x_ref[...], w_ref[...], preferred_element_type=jnp.float32)
    o_ref[...] = h * d_ref[...]


def _mid_body(a0_ref, a1_ref, hp_ref, d_ref, b_ref, w_ref, o_ref):
    d = d_ref[...]
    pre = d * (a0_ref[...] + a1_ref[...] + hp_ref[...]) + b_ref[...]
    h1 = jnp.maximum(pre, 0.0)
    o_ref[...] = jnp.dot(h1, w_ref[...], preferred_element_type=jnp.float32) * d


def _fin_body(a0_ref, a1_ref, hp_ref, d_ref, b_ref, o_ref):
    o_ref[...] = d_ref[...] * (a0_ref[...] + a1_ref[...] + hp_ref[...]) + b_ref[...]


def _row_spec(br, width):
    return pl.BlockSpec((br, width), lambda i: (i, 0))


def _full_spec(shape):
    return pl.BlockSpec(shape, lambda i: tuple(0 for _ in shape))


_BR = 1024
_GRID = NPAD // _BR


def kernel(x, edge_index, W0, b0, W1, b1):
    src = edge_index[0]
    dst = edge_index[1]
    pad_src = jnp.zeros((EPAD - E,), jnp.int32)
    pad_dst = jnp.full((EPAD - E,), NPAD - 1, jnp.int32)
    src_p = jnp.concatenate([src, pad_src])
    dst_p = jnp.concatenate([dst, pad_dst])
    comb2d = lax.bitcast_convert_type(
        (src_p | (dst_p << 16)).reshape(EPAD // ECH, ECH), jnp.float32)

    ones_h = jnp.ones((NPAD, W_SC), jnp.float32)
    deg_parts = _edge_kernel(ones_h, comb2d)
    p0 = deg_parts[0, :, 0].reshape(NPAD // 128, 128)
    p1 = deg_parts[1, :, 0].reshape(NPAD // 128, 128)

    dis2d = pl.pallas_call(
        _dis_body,
        out_shape=jax.ShapeDtypeStruct((NPAD // 128, 128), jnp.float32),
    )(p0, p1)
    dis64 = jnp.broadcast_to(dis2d.reshape(NPAD, 1), (NPAD, D_H))

    x_pad = jnp.pad(x, ((0, NPAD - N), (0, 0)))
    b0r = b0.reshape(1, D_H)
    b1r = b1.reshape(1, D_H)

    h0p = pl.pallas_call(
        _mm0_body,
        grid=(_GRID,),
        in_specs=[
            _row_spec(_BR, D_IN),
            _full_spec((D_IN, D_H)),
            _row_spec(_BR, D_H),
        ],
        out_specs=_row_spec(_BR, D_H),
        out_shape=jax.ShapeDtypeStruct((NPAD, D_H), jnp.float32),
    )(x_pad, W0, dis64)

    a0a = _edge_kernel(h0p[:, :W_SC], comb2d)
    a0b = _edge_kernel(h0p[:, W_SC:], comb2d)
    a_parts0 = jnp.concatenate([a0a, a0b], axis=-1)

    h1p = pl.pallas_call(
        _mid_body,
        grid=(_GRID,),
        in_specs=[
            _row_spec(_BR, D_H),
            _row_spec(_BR, D_H),
            _row_spec(_BR, D_H),
            _row_spec(_BR, D_H),
            _full_spec((1, D_H)),
            _full_spec((D_H, D_H)),
        ],
        out_specs=_row_spec(_BR, D_H),
        out_shape=jax.ShapeDtypeStruct((NPAD, D_H), jnp.float32),
    )(a_parts0[0], a_parts0[1], h0p, dis64, b0r, W1)

    a1a = _edge_kernel(h1p[:, :W_SC], comb2d)
    a1b = _edge_kernel(h1p[:, W_SC:], comb2d)
    a_parts1 = jnp.concatenate([a1a, a1b], axis=-1)

    out = pl.pallas_call(
        _fin_body,
        grid=(_GRID,),
        in_specs=[
            _row_spec(_BR, D_H),
            _row_spec(_BR, D_H),
            _row_spec(_BR, D_H),
            _row_spec(_BR, D_H),
            _full_spec((1, D_H)),
        ],
        out_specs=_row_spec(_BR, D_H),
        out_shape=jax.ShapeDtypeStruct((NPAD, D_H), jnp.float32),
    )(a_parts1[0], a_parts1[1], h1p, dis64, b1r)

    return out[:N]


# speedup vs baseline: 1.0006x; 1.0006x over previous
import functools

import jax
import jax.numpy as jnp
from jax import lax
from jax.experimental import pallas as pl
from jax.experimental.pallas import tpu as pltpu
from jax.experimental.pallas import tpu_sc as plsc

N = 10000
E = 320000
D_IN = 128
D_H = 64

NC = 2
NS = 16
NW = NC * NS

NPAD = 10240
ECH = 128
CPT = 80
EPAD = ECH * CPT * NW
C0 = 80
C1 = 2 * CPT - C0
CMX = max(C0, C1)
DEGW = 16
RPT = NPAD // NS
W_SC = D_H // 2

_mesh = plsc.VectorSubcoreMesh(core_axis_name="c", subcore_axis_name="s")
_sc_params = pltpu.CompilerParams(use_tc_tiling_on_sc=False, needs_layout_passes=False)


def _edge_body(h, comb2d, out, acc_sh, h_sh, combbuf, srcbuf, dstbuf, rows, zbuf, sem, sem2):
    c = lax.axis_index("c")
    s = lax.axis_index("s")
    zero16 = jnp.zeros((16,), jnp.float32)

    def fill_z(i, _):
        zbuf[i, pl.ds(0, 16)] = zero16
        zbuf[i, pl.ds(16, 16)] = zero16
        return 0

    lax.fori_loop(0, RPT // 2, fill_z, 0)
    pltpu.sync_copy(zbuf, acc_sh.at[pl.ds(s * RPT, RPT // 2)])
    pltpu.sync_copy(zbuf, acc_sh.at[pl.ds(s * RPT + RPT // 2, RPT // 2)])
    for half in range(2):
        off = s * RPT + half * (RPT // 2)
        pltpu.sync_copy(h.at[pl.ds(off, RPT // 2)], zbuf)
        pltpu.sync_copy(zbuf, h_sh.at[pl.ds(off, RPT // 2)])

    cpt = jnp.where(c == 0, C0, C1)

    @pl.when(c == 0)
    def _load_idx0():
        pltpu.sync_copy(comb2d.at[pl.ds(s * C0, C0)], combbuf.at[pl.ds(0, C0)])

    @pl.when(c == 1)
    def _load_idx1():
        pltpu.sync_copy(comb2d.at[pl.ds(NS * C0 + s * C1, C1)], combbuf.at[pl.ds(0, C1)])

    def extract(i, _):
        for k in range(8):
            v = plsc.bitcast(combbuf[i, pl.ds(16 * k, 16)], jnp.int32)
            srcbuf[i, pl.ds(16 * k, 16)] = lax.bitwise_and(v, 0xFFFF)
            dstbuf[i, pl.ds(16 * k, 16)] = lax.shift_right_logical(v, 16)
        return 0

    lax.fori_loop(0, cpt, extract, 0)
    plsc.subcore_barrier()

    pltpu.async_copy(h_sh.at[srcbuf.at[0]], rows.at[0], sem.at[0])

    def chunk(j, _):
        p = lax.rem(j, 2)

        @pl.when(j >= 1)
        def _wait_prev_scatter():
            pltpu.make_async_copy(
                rows.at[1 - p], acc_sh.at[dstbuf.at[j - 1]], sem2.at[1 - p]
            ).wait()

        @pl.when(j + 1 < cpt)
        def _start_next():
            pltpu.async_copy(h_sh.at[srcbuf.at[j + 1]], rows.at[1 - p], sem.at[1 - p])

        pltpu.make_async_copy(h_sh.at[srcbuf.at[j]], rows.at[p], sem.at[p]).wait()
        pltpu.async_copy(rows.at[p], acc_sh.at[dstbuf.at[j]], sem2.at[p], add=True)
        return 0

    lax.fori_loop(0, cpt, chunk, 0)
    plast = lax.rem(cpt - 1, 2)
    pltpu.make_async_copy(
        rows.at[plast], acc_sh.at[dstbuf.at[cpt - 1]], sem2.at[plast]
    ).wait()
    plsc.subcore_barrier()

    pltpu.sync_copy(acc_sh.at[pl.ds(s * RPT, RPT)], out.at[c, pl.ds(s * RPT, RPT)])


_edge_kernel = functools.partial(
    pl.kernel,
    out_type=jax.ShapeDtypeStruct((NC, NPAD, W_SC), jnp.float32),
    mesh=_mesh,
    scratch_types=[
        pltpu.VMEM_SHARED((NPAD, W_SC), jnp.float32),
        pltpu.VMEM_SHARED((NPAD, W_SC), jnp.float32),
        pltpu.VMEM((CMX, ECH), jnp.float32),
        pltpu.VMEM((CMX, ECH), jnp.int32),
        pltpu.VMEM((CMX, ECH), jnp.int32),
        pltpu.VMEM((2, ECH, W_SC), jnp.float32),
        pltpu.VMEM((RPT // 2, W_SC), jnp.float32),
        pltpu.SemaphoreType.DMA((2,)),
        pltpu.SemaphoreType.DMA((2,)),
    ],
    compiler_params=_sc_params,
)(_edge_body)


def _dis_body(p0_ref, p1_ref, o_ref):
    deg = p0_ref[...] + p1_ref[...] + 1.0
    o_ref[...] = lax.rsqrt(deg)


def _mm0_body(x_ref, w_ref, d_ref, o_ref):
    h = jnp.dot(x_ref[...], w_ref[...], preferred_element_type=jnp.float32)
    o_ref[...] = h * d_ref[...]


def _mid_body(a0_ref, a1_ref, hp_ref, d_ref, b_ref, w_ref, o_ref):
    d = d_ref[...]
    pre = d * (a0_ref[...] + a1_ref[...] + hp_ref[...]) + b_ref[...]
    h1 = jnp.maximum(pre, 0.0)
    o_ref[...] = jnp.dot(h1, w_ref[...], preferred_element_type=jnp.float32) * d


def _fin_body(a0_ref, a1_ref, hp_ref, d_ref, b_ref, o_ref):
    o_ref[...] = d_ref[...] * (a0_ref[...] + a1_ref[...] + hp_ref[...]) + b_ref[...]


def _row_spec(br, width):
    return pl.BlockSpec((br, width), lambda i: (i, 0))


def _full_spec(shape):
    return pl.BlockSpec(shape, lambda i: tuple(0 for _ in shape))


_BR = 1024
_GRID = NPAD // _BR


def kernel(x, edge_index, W0, b0, W1, b1):
    src = edge_index[0]
    dst = edge_index[1]
    pad_src = jnp.zeros((EPAD - E,), jnp.int32)
    pad_dst = jnp.full((EPAD - E,), NPAD - 1, jnp.int32)
    src_p = jnp.concatenate([src, pad_src])
    dst_p = jnp.concatenate([dst, pad_dst])
    comb2d = lax.bitcast_convert_type(
        (src_p | (dst_p << 16)).reshape(EPAD // ECH, ECH), jnp.float32)

    ones_h = jnp.ones((NPAD, W_SC), jnp.float32)
    deg_parts = _edge_kernel(ones_h, comb2d)
    p0 = deg_parts[0, :, 0].reshape(NPAD // 128, 128)
    p1 = deg_parts[1, :, 0].reshape(NPAD // 128, 128)

    dis2d = pl.pallas_call(
        _dis_body,
        out_shape=jax.ShapeDtypeStruct((NPAD // 128, 128), jnp.float32),
    )(p0, p1)
    dis64 = jnp.broadcast_to(dis2d.reshape(NPAD, 1), (NPAD, D_H))

    x_pad = jnp.pad(x, ((0, NPAD - N), (0, 0)))
    b0r = b0.reshape(1, D_H)
    b1r = b1.reshape(1, D_H)

    h0p = pl.pallas_call(
        _mm0_body,
        grid=(_GRID,),
        in_specs=[
            _row_spec(_BR, D_IN),
            _full_spec((D_IN, D_H)),
            _row_spec(_BR, D_H),
        ],
        out_specs=_row_spec(_BR, D_H),
        out_shape=jax.ShapeDtypeStruct((NPAD, D_H), jnp.float32),
    )(x_pad, W0, dis64)

    a0a = _edge_kernel(h0p[:, :W_SC], comb2d)
    a0b = _edge_kernel(h0p[:, W_SC:], comb2d)
    a_parts0 = jnp.concatenate([a0a, a0b], axis=-1)

    h1p = pl.pallas_call(
        _mid_body,
        grid=(_GRID,),
        in_specs=[
            _row_spec(_BR, D_H),
            _row_spec(_BR, D_H),
            _row_spec(_BR, D_H),
            _row_spec(_BR, D_H),
            _full_spec((1, D_H)),
            _full_spec((D_H, D_H)),
        ],
        out_specs=_row_spec(_BR, D_H),
        out_shape=jax.ShapeDtypeStruct((NPAD, D_H), jnp.float32),
    )(a_parts0[0], a_parts0[1], h0p, dis64, b0r, W1)

    a1a = _edge_kernel(h1p[:, :W_SC], comb2d)
    a1b = _edge_kernel(h1p[:, W_SC:], comb2d)
    a_parts1 = jnp.concatenate([a1a, a1b], axis=-1)

    out = pl.pallas_call(
        _fin_body,
        grid=(_GRID,),
        in_specs=[
            _row_spec(_BR, D_H),
            _row_spec(_BR, D_H),
            _row_spec(_BR, D_H),
            _row_spec(_BR, D_H),
            _full_spec((1, D_H)),
        ],
        out_specs=_row_spec(_BR, D_H),
        out_shape=jax.ShapeDtypeStruct((NPAD, D_H), jnp.float32),
    )(a_parts1[0], a_parts1[1], h1p, dis64, b1r)

    return out[:N]


# speedup vs baseline: 1.0074x; 1.0068x over previous
import functools

import jax
import jax.numpy as jnp
from jax import lax
from jax.experimental import pallas as pl
from jax.experimental.pallas import tpu as pltpu
from jax.experimental.pallas import tpu_sc as plsc

N = 10000
E = 320000
D_IN = 128
D_H = 64

NC = 2
NS = 16
NW = NC * NS

NPAD = 10240
ECH = 128
CPT = 160
EPAD = ECH * CPT * NS
RPT = NPAD // NS
W_SC = D_H // 2

_mesh = plsc.VectorSubcoreMesh(core_axis_name="c", subcore_axis_name="s")
_sc_params = pltpu.CompilerParams(use_tc_tiling_on_sc=False, needs_layout_passes=False)


def _edge_body(ha, hb, comb2d, out, acc_sh, h_sh, combbuf, srcbuf, dstbuf, rows, zbuf, sem, sem2):
    c = lax.axis_index("c")
    s = lax.axis_index("s")
    zero16 = jnp.zeros((16,), jnp.float32)

    def fill_z(i, _):
        zbuf[i, pl.ds(0, 16)] = zero16
        zbuf[i, pl.ds(16, 16)] = zero16
        return 0

    lax.fori_loop(0, RPT // 2, fill_z, 0)
    pltpu.sync_copy(zbuf, acc_sh.at[pl.ds(s * RPT, RPT // 2)])
    pltpu.sync_copy(zbuf, acc_sh.at[pl.ds(s * RPT + RPT // 2, RPT // 2)])
    for half in range(2):
        off = s * RPT + half * (RPT // 2)

        @pl.when(c == 0)
        def _stage_a():
            pltpu.sync_copy(ha.at[pl.ds(off, RPT // 2)], zbuf)

        @pl.when(c == 1)
        def _stage_b():
            pltpu.sync_copy(hb.at[pl.ds(off, RPT // 2)], zbuf)

        pltpu.sync_copy(zbuf, h_sh.at[pl.ds(off, RPT // 2)])

    cpt = CPT
    pltpu.sync_copy(comb2d.at[pl.ds(s * CPT, CPT)], combbuf)

    def extract(i, _):
        for k in range(8):
            v = plsc.bitcast(combbuf[i, pl.ds(16 * k, 16)], jnp.int32)
            srcbuf[i, pl.ds(16 * k, 16)] = lax.bitwise_and(v, 0xFFFF)
            dstbuf[i, pl.ds(16 * k, 16)] = lax.shift_right_logical(v, 16)
        return 0

    lax.fori_loop(0, cpt, extract, 0)
    plsc.subcore_barrier()

    pltpu.async_copy(h_sh.at[srcbuf.at[0]], rows.at[0], sem.at[0])

    def chunk(j, _):
        p = lax.rem(j, 2)

        @pl.when(j >= 1)
        def _wait_prev_scatter():
            pltpu.make_async_copy(
                rows.at[1 - p], acc_sh.at[dstbuf.at[j - 1]], sem2.at[1 - p]
            ).wait()

        @pl.when(j + 1 < cpt)
        def _start_next():
            pltpu.async_copy(h_sh.at[srcbuf.at[j + 1]], rows.at[1 - p], sem.at[1 - p])

        pltpu.make_async_copy(h_sh.at[srcbuf.at[j]], rows.at[p], sem.at[p]).wait()
        pltpu.async_copy(rows.at[p], acc_sh.at[dstbuf.at[j]], sem2.at[p], add=True)
        return 0

    lax.fori_loop(0, cpt, chunk, 0)
    plast = lax.rem(cpt - 1, 2)
    pltpu.make_async_copy(
        rows.at[plast], acc_sh.at[dstbuf.at[cpt - 1]], sem2.at[plast]
    ).wait()
    plsc.subcore_barrier()

    pltpu.sync_copy(acc_sh.at[pl.ds(s * RPT, RPT)], out.at[c, pl.ds(s * RPT, RPT)])


_edge_kernel = functools.partial(
    pl.kernel,
    out_type=jax.ShapeDtypeStruct((NC, NPAD, W_SC), jnp.float32),
    mesh=_mesh,
    scratch_types=[
        pltpu.VMEM_SHARED((NPAD, W_SC), jnp.float32),
        pltpu.VMEM_SHARED((NPAD, W_SC), jnp.float32),
        pltpu.VMEM((CPT, ECH), jnp.float32),
        pltpu.VMEM((CPT, ECH), jnp.int32),
        pltpu.VMEM((CPT, ECH), jnp.int32),
        pltpu.VMEM((2, ECH, W_SC), jnp.float32),
        pltpu.VMEM((RPT // 2, W_SC), jnp.float32),
        pltpu.SemaphoreType.DMA((2,)),
        pltpu.SemaphoreType.DMA((2,)),
    ],
    compiler_params=_sc_params,
)(_edge_body)


def _dis_body(p0_ref, o_ref):
    o_ref[...] = lax.rsqrt(p0_ref[...] + 1.0)


def _mm0_body(x_ref, w_ref, d_ref, o_ref):
    h = jnp.dot(x_ref[...], w_ref[...], preferred_element_type=jnp.float32)
    o_ref[...] = h * d_ref[...]


def _mid_body(aa_ref, ab_ref, hp_ref, d_ref, b_ref, w_ref, o_ref):
    d = d_ref[...]
    a = jnp.concatenate([aa_ref[...], ab_ref[...]], axis=1)
    pre = d * (a + hp_ref[...]) + b_ref[...]
    h1 = jnp.maximum(pre, 0.0)
    o_ref[...] = jnp.dot(h1, w_ref[...], preferred_element_type=jnp.float32) * d


def _fin_body(aa_ref, ab_ref, hp_ref, d_ref, b_ref, o_ref):
    a = jnp.concatenate([aa_ref[...], ab_ref[...]], axis=1)
    o_ref[...] = d_ref[...] * a + hp_ref[...] * d_ref[...] + b_ref[...]


def _row_spec(br, width):
    return pl.BlockSpec((br, width), lambda i: (i, 0))


def _full_spec(shape):
    return pl.BlockSpec(shape, lambda i: tuple(0 for _ in shape))


_BR = 1024
_GRID = NPAD // _BR


def kernel(x, edge_index, W0, b0, W1, b1):
    src = edge_index[0]
    dst = edge_index[1]
    pad_src = jnp.zeros((EPAD - E,), jnp.int32)
    pad_dst = jnp.full((EPAD - E,), NPAD - 1, jnp.int32)
    src_p = jnp.concatenate([src, pad_src])
    dst_p = jnp.concatenate([dst, pad_dst])
    comb2d = lax.bitcast_convert_type(
        (src_p | (dst_p << 16)).reshape(EPAD // ECH, ECH), jnp.float32)

    ones_h = jnp.ones((NPAD, W_SC), jnp.float32)
    deg_full = _edge_kernel(ones_h, ones_h, comb2d)
    p0 = deg_full[0, :, 0].reshape(NPAD // 128, 128)

    dis2d = pl.pallas_call(
        _dis_body,
        out_shape=jax.ShapeDtypeStruct((NPAD // 128, 128), jnp.float32),
    )(p0)
    dis64 = jnp.broadcast_to(dis2d.reshape(NPAD, 1), (NPAD, D_H))

    x_pad = jnp.pad(x, ((0, NPAD - N), (0, 0)))
    b0r = b0.reshape(1, D_H)
    b1r = b1.reshape(1, D_H)

    h0p = pl.pallas_call(
        _mm0_body,
        grid=(_GRID,),
        in_specs=[
            _row_spec(_BR, D_IN),
            _full_spec((D_IN, D_H)),
            _row_spec(_BR, D_H),
        ],
        out_specs=_row_spec(_BR, D_H),
        out_shape=jax.ShapeDtypeStruct((NPAD, D_H), jnp.float32),
    )(x_pad, W0, dis64)

    a0 = _edge_kernel(h0p[:, :W_SC], h0p[:, W_SC:], comb2d)

    h1p = pl.pallas_call(
        _mid_body,
        grid=(_GRID,),
        in_specs=[
            _row_spec(_BR, W_SC),
            _row_spec(_BR, W_SC),
            _row_spec(_BR, D_H),
            _row_spec(_BR, D_H),
            _full_spec((1, D_H)),
            _full_spec((D_H, D_H)),
        ],
        out_specs=_row_spec(_BR, D_H),
        out_shape=jax.ShapeDtypeStruct((NPAD, D_H), jnp.float32),
    )(a0[0], a0[1], h0p, dis64, b0r, W1)

    a1 = _edge_kernel(h1p[:, :W_SC], h1p[:, W_SC:], comb2d)

    out = pl.pallas_call(
        _fin_body,
        grid=(_GRID,),
        in_specs=[
            _row_spec(_BR, W_SC),
            _row_spec(_BR, W_SC),
            _row_spec(_BR, D_H),
            _row_spec(_BR, D_H),
            _full_spec((1, D_H)),
        ],
        out_specs=_row_spec(_BR, D_H),
        out_shape=jax.ShapeDtypeStruct((NPAD, D_H), jnp.float32),
    )(a1[0], a1[1], h1p, dis64, b1r)

    return out[:N]


# speedup vs baseline: 1.0217x; 1.0142x over previous
import functools

import jax
import jax.numpy as jnp
from jax import lax
from jax.experimental import pallas as pl
from jax.experimental.pallas import tpu as pltpu
from jax.experimental.pallas import tpu_sc as plsc

N = 10000
E = 320000
D_IN = 128
D_H = 64

NC = 2
NS = 16
NW = NC * NS

NPAD = 10240
ECH = 128
CPT = 160
EPAD = ECH * CPT * NS
RPT = NPAD // NS
W_SC = D_H // 2

_mesh = plsc.VectorSubcoreMesh(core_axis_name="c", subcore_axis_name="s")
_sc_params = pltpu.CompilerParams(use_tc_tiling_on_sc=False, needs_layout_passes=False)


def _edge_body(ha, hb, comb2d, out, acc_sh, h_sh, combbuf, srcbuf, dstbuf, rows, zbuf, sem, sem2):
    c = lax.axis_index("c")
    s = lax.axis_index("s")
    zero16 = jnp.zeros((16,), jnp.float32)

    def fill_z(i, _):
        zbuf[i, pl.ds(0, 16)] = zero16
        zbuf[i, pl.ds(16, 16)] = zero16
        return 0

    lax.fori_loop(0, RPT // 2, fill_z, 0)
    pltpu.sync_copy(zbuf, acc_sh.at[pl.ds(s * RPT, RPT // 2)])
    pltpu.sync_copy(zbuf, acc_sh.at[pl.ds(s * RPT + RPT // 2, RPT // 2)])
    for half in range(2):
        off = s * RPT + half * (RPT // 2)

        @pl.when(c == 0)
        def _stage_a():
            pltpu.sync_copy(ha.at[pl.ds(off, RPT // 2)], zbuf)

        @pl.when(c == 1)
        def _stage_b():
            pltpu.sync_copy(hb.at[pl.ds(off, RPT // 2)], zbuf)

        pltpu.sync_copy(zbuf, h_sh.at[pl.ds(off, RPT // 2)])

    cpt = CPT
    pltpu.sync_copy(comb2d.at[pl.ds(s * CPT, CPT)], combbuf)

    def extract(i, _):
        for k in range(8):
            v = plsc.bitcast(combbuf[i, pl.ds(16 * k, 16)], jnp.int32)
            srcbuf[i, pl.ds(16 * k, 16)] = lax.bitwise_and(v, 0xFFFF)
            dstbuf[i, pl.ds(16 * k, 16)] = lax.shift_right_logical(v, 16)
        return 0

    lax.fori_loop(0, cpt, extract, 0)
    plsc.subcore_barrier()

    pltpu.async_copy(h_sh.at[srcbuf.at[0]], rows.at[0], sem.at[0])

    def chunk(j, _):
        p = lax.rem(j, 2)

        @pl.when(j >= 1)
        def _wait_prev_scatter():
            pltpu.make_async_copy(
                rows.at[1 - p], acc_sh.at[dstbuf.at[j - 1]], sem2.at[1 - p]
            ).wait()

        @pl.when(j + 1 < cpt)
        def _start_next():
            pltpu.async_copy(h_sh.at[srcbuf.at[j + 1]], rows.at[1 - p], sem.at[1 - p])

        pltpu.make_async_copy(h_sh.at[srcbuf.at[j]], rows.at[p], sem.at[p]).wait()
        pltpu.async_copy(rows.at[p], acc_sh.at[dstbuf.at[j]], sem2.at[p], add=True)
        return 0

    lax.fori_loop(0, cpt, chunk, 0)
    plast = lax.rem(cpt - 1, 2)
    pltpu.make_async_copy(
        rows.at[plast], acc_sh.at[dstbuf.at[cpt - 1]], sem2.at[plast]
    ).wait()
    plsc.subcore_barrier()

    pltpu.sync_copy(acc_sh.at[pl.ds(s * RPT, RPT)], out.at[c, pl.ds(s * RPT, RPT)])


_edge_kernel = functools.partial(
    pl.kernel,
    out_type=jax.ShapeDtypeStruct((NC, NPAD, W_SC), jnp.float32),
    mesh=_mesh,
    scratch_types=[
        pltpu.VMEM_SHARED((NPAD, W_SC), jnp.float32),
        pltpu.VMEM_SHARED((NPAD, W_SC), jnp.float32),
        pltpu.VMEM((CPT, ECH), jnp.float32),
        pltpu.VMEM((CPT, ECH), jnp.int32),
        pltpu.VMEM((CPT, ECH), jnp.int32),
        pltpu.VMEM((2, ECH, W_SC), jnp.float32),
        pltpu.VMEM((RPT // 2, W_SC), jnp.float32),
        pltpu.SemaphoreType.DMA((2,)),
        pltpu.SemaphoreType.DMA((2,)),
    ],
    compiler_params=_sc_params,
)(_edge_body)


def _mm0_body(x_ref, w_ref, dc_ref, o_ref):
    d = lax.rsqrt(dc_ref[...] + 1.0)
    h = jnp.dot(x_ref[...], w_ref[...], preferred_element_type=jnp.float32)
    o_ref[...] = h * d


def _mid_body(aa_ref, ab_ref, hp_ref, dc_ref, b_ref, w_ref, o_ref):
    d = lax.rsqrt(dc_ref[...] + 1.0)
    a = jnp.concatenate([aa_ref[...], ab_ref[...]], axis=1)
    pre = d * (a + hp_ref[...]) + b_ref[...]
    h1 = jnp.maximum(pre, 0.0)
    o_ref[...] = jnp.dot(h1, w_ref[...], preferred_element_type=jnp.float32) * d


def _fin_body(aa_ref, ab_ref, hp_ref, dc_ref, b_ref, o_ref):
    d = lax.rsqrt(dc_ref[...] + 1.0)
    a = jnp.concatenate([aa_ref[...], ab_ref[...]], axis=1)
    o_ref[...] = d * a + hp_ref[...] * d + b_ref[...]


def _row_spec(br, width):
    return pl.BlockSpec((br, width), lambda i: (i, 0))


def _full_spec(shape):
    return pl.BlockSpec(shape, lambda i: tuple(0 for _ in shape))


_BR = 1024
_GRID = NPAD // _BR


def kernel(x, edge_index, W0, b0, W1, b1):
    src = edge_index[0]
    dst = edge_index[1]
    pad_src = jnp.zeros((EPAD - E,), jnp.int32)
    pad_dst = jnp.full((EPAD - E,), NPAD - 1, jnp.int32)
    src_p = jnp.concatenate([src, pad_src])
    dst_p = jnp.concatenate([dst, pad_dst])
    comb2d = lax.bitcast_convert_type(
        (src_p | (dst_p << 16)).reshape(EPAD // ECH, ECH), jnp.float32)

    ones_h = jnp.ones((NPAD, W_SC), jnp.float32)
    deg_full = _edge_kernel(ones_h, ones_h, comb2d)
    degcol = deg_full[0, :, 0:1]

    x_pad = jnp.pad(x, ((0, NPAD - N), (0, 0)))
    b0r = b0.reshape(1, D_H)
    b1r = b1.reshape(1, D_H)

    h0p = pl.pallas_call(
        _mm0_body,
        grid=(_GRID,),
        in_specs=[
            _row_spec(_BR, D_IN),
            _full_spec((D_IN, D_H)),
            _row_spec(_BR, 1),
        ],
        out_specs=_row_spec(_BR, D_H),
        out_shape=jax.ShapeDtypeStruct((NPAD, D_H), jnp.float32),
    )(x_pad, W0, degcol)

    a0 = _edge_kernel(h0p[:, :W_SC], h0p[:, W_SC:], comb2d)

    h1p = pl.pallas_call(
        _mid_body,
        grid=(_GRID,),
        in_specs=[
            _row_spec(_BR, W_SC),
            _row_spec(_BR, W_SC),
            _row_spec(_BR, D_H),
            _row_spec(_BR, 1),
            _full_spec((1, D_H)),
            _full_spec((D_H, D_H)),
        ],
        out_specs=_row_spec(_BR, D_H),
        out_shape=jax.ShapeDtypeStruct((NPAD, D_H), jnp.float32),
    )(a0[0], a0[1], h0p, degcol, b0r, W1)

    a1 = _edge_kernel(h1p[:, :W_SC], h1p[:, W_SC:], comb2d)

    out = pl.pallas_call(
        _fin_body,
        grid=(_GRID,),
        in_specs=[
            _row_spec(_BR, W_SC),
            _row_spec(_BR, W_SC),
            _row_spec(_BR, D_H),
            _row_spec(_BR, 1),
            _full_spec((1, D_H)),
        ],
        out_specs=_row_spec(_BR, D_H),
        out_shape=jax.ShapeDtypeStruct((NPAD, D_H), jnp.float32),
    )(a1[0], a1[1], h1p, degcol, b1r)

    return out[:N]
